# baseline (speedup 1.0000x reference)
import jax
import jax.numpy as jnp
from jax.experimental import pallas as pl
from jax.experimental.pallas import tpu as pltpu

D_INT = 32
D_MODEL = 1024
N_TOK = 4096
O_TILE = 128
I_CHUNK = 128


def _fused_body(x_ref, Vd_ref, W0_ref, Rw_ref, Rb_ref, b0_ref, out_ref):
    i = pl.program_id(1)
    rows = Rw_ref[...].reshape(O_TILE * I_CHUNK, D_INT)
    V128 = Vd_ref[...]

    o_idx = jax.lax.broadcasted_iota(jnp.int32, (O_TILE, 1, O_TILE), 0)
    c_idx = jax.lax.broadcasted_iota(jnp.int32, (O_TILE, 1, O_TILE), 2)
    eye3 = (o_idx == c_idx).astype(jnp.float32)

    Wdup = jax.lax.dot_general(rows, V128, (((1,), (0,)), ((), ())))
    D3 = Wdup.reshape(O_TILE, I_CHUNK, O_TILE)
    WT_chunk = jnp.sum(D3 * eye3, axis=0)

    xs = x_ref[:, pl.ds(i * I_CHUNK, I_CHUNK)]
    acc = jax.lax.dot_general(xs, WT_chunk, (((1,), (0,)), ((), ())))
    acc = acc + jax.lax.dot_general(xs, W0_ref[...], (((1,), (1,)), ((), ())))

    @pl.when(i == 0)
    def _():
        m_idx = jax.lax.broadcasted_iota(jnp.int32, (O_TILE, O_TILE), 0)
        j_idx = jax.lax.broadcasted_iota(jnp.int32, (O_TILE, O_TILE), 1)
        eye2 = (m_idx == j_idx).astype(jnp.float32)
        bdup = jax.lax.dot_general(Rb_ref[...], V128, (((1,), (0,)), ((), ())))
        bias = jnp.sum(bdup * eye2, axis=0, keepdims=True) + b0_ref[...]
        out_ref[...] = acc + bias

    @pl.when(i != 0)
    def _():
        out_ref[...] = out_ref[...] + acc


def kernel(x, V, W0, b0, R_w, R_b):
    V128 = jnp.tile(V, (1, O_TILE))
    b02 = b0.reshape(1, D_MODEL)

    return pl.pallas_call(
        _fused_body,
        grid=(D_MODEL // O_TILE, D_MODEL // I_CHUNK),
        in_specs=[
            pl.BlockSpec((N_TOK, D_MODEL), lambda o, i: (0, 0)),
            pl.BlockSpec((D_INT, O_TILE), lambda o, i: (0, 0)),
            pl.BlockSpec((O_TILE, I_CHUNK), lambda o, i: (o, i)),
            pl.BlockSpec((O_TILE, I_CHUNK, D_INT), lambda o, i: (o, i, 0)),
            pl.BlockSpec((O_TILE, D_INT), lambda o, i: (o, 0)),
            pl.BlockSpec((1, O_TILE), lambda o, i: (0, o)),
        ],
        out_specs=pl.BlockSpec((N_TOK, O_TILE), lambda o, i: (0, o)),
        out_shape=jax.ShapeDtypeStruct((N_TOK, D_MODEL), jnp.float32),
        compiler_params=pltpu.CompilerParams(
            dimension_semantics=("arbitrary", "arbitrary"),
        ),
    )(x, V128, W0, R_w, R_b, b02)
